# baseline (speedup 1.0000x reference)
import functools
import math

import jax
import jax.numpy as jnp
from jax import lax
from jax.experimental import pallas as pl
from jax.experimental.pallas import tpu as pltpu
from jax.experimental.pallas import tpu_sc as plsc

N = 10000
E = 320000
D = 128
L = 5
ALPHA = 0.1
THETA = 0.5

NC = 2
NS = 16
NW = NC * NS
CHUNK = 64
NCHUNK = 160
EPT = NCHUNK * CHUNK
E_PAD = EPT * NW
ROWS_PER_TILE = 640
N_PAD = NS * ROWS_PER_TILE


def _sc_agg_body(z_hbm, pk_hbm, out_hbm,
                 pk_v, src_c, dst_c, rows_a, rows_b, agg_sh, sem_a, sem_b):
    c = lax.axis_index("c")
    s = lax.axis_index("s")
    wid = c * NS + s

    pltpu.sync_copy(pk_hbm.at[wid], pk_v)

    def unpack(j, b):
        def upk(k, _):
            v = pk_v[j, pl.ds(k * 16, 16)]
            src_c[b, pl.ds(k * 16, 16)] = v & 0xFFFF
            dst_c[b, pl.ds(k * 16, 16)] = lax.shift_right_logical(v, 16)
            return 0
        lax.fori_loop(0, CHUNK // 16, upk, 0)

    zero16 = jnp.zeros((16,), jnp.float32)

    def zrow(i, _):
        def zcol(k, _):
            rows_a[i, pl.ds(k * 16, 16)] = zero16
            return 0
        return lax.fori_loop(0, D // 16, zcol, 0)

    lax.fori_loop(0, CHUNK, zrow, 0)

    row_base = s * ROWS_PER_TILE
    for r in range(ROWS_PER_TILE // CHUNK):
        pltpu.sync_copy(rows_a, agg_sh.at[pl.ds(row_base + r * CHUNK, CHUNK)])
    plsc.subcore_barrier()

    nhalf = NCHUNK // 2
    unpack(0, 0)
    pltpu.async_copy(z_hbm.at[src_c.at[0]], rows_a, sem_a)

    def body(i, _):
        j0 = 2 * i
        unpack(j0 + 1, 1)
        pltpu.make_async_copy(z_hbm.at[src_c.at[0]], rows_a, sem_a).wait()
        pltpu.async_copy(z_hbm.at[src_c.at[1]], rows_b, sem_b)
        pltpu.sync_copy(rows_a, agg_sh.at[dst_c.at[0]], add=True)
        pltpu.make_async_copy(z_hbm.at[src_c.at[1]], rows_b, sem_b).wait()

        @pl.when(i + 1 < nhalf)
        def _():
            unpack(j0 + 2, 0)
            pltpu.async_copy(z_hbm.at[src_c.at[0]], rows_a, sem_a)

        pltpu.sync_copy(rows_b, agg_sh.at[dst_c.at[1]], add=True)
        return 0

    lax.fori_loop(0, nhalf, body, 0)
    plsc.subcore_barrier()

    pltpu.sync_copy(agg_sh.at[pl.ds(row_base, ROWS_PER_TILE)],
                    out_hbm.at[c, pl.ds(row_base, ROWS_PER_TILE)])


@functools.cache
def _get_sc_agg():
    return functools.partial(
        pl.kernel,
        out_type=jax.ShapeDtypeStruct((NC, N_PAD, D), jnp.float32),
        mesh=plsc.VectorSubcoreMesh(core_axis_name="c", subcore_axis_name="s",
                                    num_cores=NC, num_subcores=NS),
        scratch_types=[
            pltpu.VMEM((NCHUNK, CHUNK), jnp.int32),
            pltpu.VMEM((2, CHUNK), jnp.int32),
            pltpu.VMEM((2, CHUNK), jnp.int32),
            pltpu.VMEM((CHUNK, D), jnp.float32),
            pltpu.VMEM((CHUNK, D), jnp.float32),
            pltpu.VMEM_SHARED((N_PAD, D), jnp.float32),
            pltpu.SemaphoreType.DMA,
            pltpu.SemaphoreType.DMA,
        ],
        compiler_params=pltpu.CompilerParams(use_tc_tiling_on_sc=False),
    )(_sc_agg_body)


def _sc_agg(z, pk):
    return _get_sc_agg()(z, pk)


def _row_mask():
    rows = lax.broadcasted_iota(jnp.int32, (N_PAD, 1), 0)
    return rows < N


def _lin0_body(x_ref, w_ref, b_ref, o_ref):
    z = jnp.dot(x_ref[...], w_ref[...], preferred_element_type=jnp.float32)
    z = z + b_ref[...]
    o_ref[...] = jnp.where(_row_mask(), z, 0.0)


_lin0 = pl.pallas_call(
    _lin0_body,
    out_shape=jax.ShapeDtypeStruct((N_PAD, D), jnp.float32),
)


def _gcn_update(p_ref, x0_ref, w_ref, bl):
    agg = p_ref[0] + p_ref[1]
    out = agg * (1.0 - ALPHA) + ALPHA * x0_ref[...]
    return out * (1.0 - bl) + bl * jnp.dot(
        out, w_ref[...], preferred_element_type=jnp.float32)


def _layer_body(p_ref, x0_ref, w_ref, g_ref, bta_ref, u_ref, z_ref, *, bl):
    u = _gcn_update(p_ref, x0_ref, w_ref, bl)
    u_ref[...] = u
    mean = jnp.sum(u, axis=0, keepdims=True) * (1.0 / N)
    d = u - mean
    mask = _row_mask()
    d = jnp.where(mask, d, 0.0)
    var = jnp.sum(d * d, axis=0, keepdims=True) * (1.0 / N)
    zn = d * lax.rsqrt(var + 1e-5) * g_ref[...] + bta_ref[...]
    zn = jnp.maximum(zn, 0.0)
    z_ref[...] = jnp.where(mask, zn, 0.0)


def _make_layer(bl):
    return pl.pallas_call(
        functools.partial(_layer_body, bl=bl),
        out_shape=(jax.ShapeDtypeStruct((N_PAD, D), jnp.float32),
                   jax.ShapeDtypeStruct((N_PAD, D), jnp.float32)),
    )


def _jk_body(p_ref, x0_ref, w_ref, z0_ref, z1_ref, z2_ref, wjk_ref, bjk_ref,
             z_ref, *, bl):
    u3 = _gcn_update(p_ref, x0_ref, w_ref, bl)
    acc = jnp.dot(z0_ref[...], wjk_ref[0], preferred_element_type=jnp.float32)
    acc += jnp.dot(z1_ref[...], wjk_ref[1], preferred_element_type=jnp.float32)
    acc += jnp.dot(z2_ref[...], wjk_ref[2], preferred_element_type=jnp.float32)
    acc += jnp.dot(u3, wjk_ref[3], preferred_element_type=jnp.float32)
    acc += bjk_ref[...]
    z_ref[...] = jnp.where(_row_mask(), acc, 0.0)


def _make_jk(bl):
    return pl.pallas_call(
        functools.partial(_jk_body, bl=bl),
        out_shape=jax.ShapeDtypeStruct((N_PAD, D), jnp.float32),
    )


def _final_body(p_ref, x0_ref, w_ref, o_ref, *, bl):
    u = _gcn_update(p_ref, x0_ref, w_ref, bl)
    o_ref[...] = u[:N]


def _make_final(bl):
    return pl.pallas_call(
        functools.partial(_final_body, bl=bl),
        out_shape=jax.ShapeDtypeStruct((N, D), jnp.float32),
    )


def kernel(x, edge_index, W0, b0, Wc, W_jk, b_jk, gamma, beta):
    src = edge_index[0]
    dst = edge_index[1]
    pad = N + (jnp.arange(E_PAD - E, dtype=jnp.int32) % (N_PAD - N))
    src_p = jnp.concatenate([src, pad])
    dst_p = jnp.concatenate([dst, pad])
    pk = ((dst_p << 16) | src_p).reshape(NW, NCHUNK, CHUNK)

    x_p = jnp.zeros((N_PAD, D), jnp.float32).at[:N].set(x)
    b0r = b0.reshape(1, D)
    bjkr = b_jk.reshape(1, D)
    wjk = W_jk.reshape(4, D, D)

    z = _lin0(x_p, W0, b0r)
    x0 = z
    zs = []
    for i in range(L):
        bl = float(math.log(THETA / (i + 1) + 1.0))
        parts = _sc_agg(z, pk)
        if i < L - 2:
            u, z = _make_layer(bl)(parts, x0, Wc[i],
                                   gamma[i].reshape(1, D),
                                   beta[i].reshape(1, D))
            zs.append(u)
        elif i == L - 2:
            z = _make_jk(bl)(parts, x0, Wc[i], zs[0], zs[1], zs[2],
                             wjk, bjkr)
        else:
            z = _make_final(bl)(parts, x0, Wc[i])
    return z


# speedup vs baseline: 1.3209x; 1.3209x over previous
import functools
import math

import jax
import jax.numpy as jnp
from jax import lax
from jax.experimental import pallas as pl
from jax.experimental.pallas import tpu as pltpu
from jax.experimental.pallas import tpu_sc as plsc

N = 10000
E = 320000
D = 128
L = 5
ALPHA = 0.1
THETA = 0.5

NC = 2
NS = 16
NW = NC * NS
CHUNK = 64
NCHUNK = 160
EPT = NCHUNK * CHUNK
E_PAD = EPT * NW
ROWS_PER_TILE = 640
N_PAD = NS * ROWS_PER_TILE


def _sc_agg_body(z_hbm, pk_hbm, out_hbm,
                 pk_v, src_c, dst_c, rows_a, rows_b, agg_sh, sem_a, sem_b):
    c = lax.axis_index("c")
    s = lax.axis_index("s")
    wid = c * NS + s

    pltpu.sync_copy(pk_hbm.at[wid], pk_v)

    def unpack(j, b):
        def upk(k, _):
            v = pk_v[j, pl.ds(k * 16, 16)]
            src_c[b, pl.ds(k * 16, 16)] = v & 0xFFFF
            dst_c[b, pl.ds(k * 16, 16)] = lax.shift_right_logical(v, 16)
            return 0
        lax.fori_loop(0, CHUNK // 16, upk, 0)

    zero16 = jnp.zeros((16,), jnp.float32)

    def zrow(i, _):
        def zcol(k, _):
            rows_a[i, pl.ds(k * 16, 16)] = zero16
            return 0
        return lax.fori_loop(0, D // 16, zcol, 0)

    lax.fori_loop(0, CHUNK, zrow, 0)

    row_base = s * ROWS_PER_TILE
    for r in range(ROWS_PER_TILE // CHUNK):
        pltpu.sync_copy(rows_a, agg_sh.at[pl.ds(row_base + r * CHUNK, CHUNK)])
    plsc.subcore_barrier()

    nhalf = NCHUNK // 2
    unpack(0, 0)
    pltpu.async_copy(z_hbm.at[src_c.at[0]], rows_a, sem_a)
    unpack(1, 1)
    pltpu.async_copy(z_hbm.at[src_c.at[1]], rows_b, sem_b)

    def body(i, _):
        j0 = 2 * i
        pltpu.make_async_copy(z_hbm.at[src_c.at[0]], rows_a, sem_a).wait()
        pltpu.sync_copy(rows_a, agg_sh.at[dst_c.at[0]], add=True)

        @pl.when(i + 1 < nhalf)
        def _():
            unpack(j0 + 2, 0)
            pltpu.async_copy(z_hbm.at[src_c.at[0]], rows_a, sem_a)

        pltpu.make_async_copy(z_hbm.at[src_c.at[1]], rows_b, sem_b).wait()
        pltpu.sync_copy(rows_b, agg_sh.at[dst_c.at[1]], add=True)

        @pl.when(i + 1 < nhalf)
        def _():
            unpack(j0 + 3, 1)
            pltpu.async_copy(z_hbm.at[src_c.at[1]], rows_b, sem_b)

        return 0

    lax.fori_loop(0, nhalf, body, 0)
    plsc.subcore_barrier()

    pltpu.sync_copy(agg_sh.at[pl.ds(row_base, ROWS_PER_TILE)],
                    out_hbm.at[c, pl.ds(row_base, ROWS_PER_TILE)])


@functools.cache
def _get_sc_agg():
    return functools.partial(
        pl.kernel,
        out_type=jax.ShapeDtypeStruct((NC, N_PAD, D), jnp.float32),
        mesh=plsc.VectorSubcoreMesh(core_axis_name="c", subcore_axis_name="s",
                                    num_cores=NC, num_subcores=NS),
        scratch_types=[
            pltpu.VMEM((NCHUNK, CHUNK), jnp.int32),
            pltpu.VMEM((2, CHUNK), jnp.int32),
            pltpu.VMEM((2, CHUNK), jnp.int32),
            pltpu.VMEM((CHUNK, D), jnp.float32),
            pltpu.VMEM((CHUNK, D), jnp.float32),
            pltpu.VMEM_SHARED((N_PAD, D), jnp.float32),
            pltpu.SemaphoreType.DMA,
            pltpu.SemaphoreType.DMA,
        ],
        compiler_params=pltpu.CompilerParams(use_tc_tiling_on_sc=False),
    )(_sc_agg_body)


def _sc_agg(z, pk):
    return _get_sc_agg()(z, pk)


def _row_mask():
    rows = lax.broadcasted_iota(jnp.int32, (N_PAD, 1), 0)
    return rows < N


def _lin0_body(x_ref, w_ref, b_ref, o_ref):
    z = jnp.dot(x_ref[...], w_ref[...], preferred_element_type=jnp.float32)
    z = z + b_ref[...]
    o_ref[...] = jnp.where(_row_mask(), z, 0.0)


_lin0 = pl.pallas_call(
    _lin0_body,
    out_shape=jax.ShapeDtypeStruct((N_PAD, D), jnp.float32),
)


def _gcn_update(p_ref, x0_ref, w_ref, bl):
    agg = p_ref[0] + p_ref[1]
    out = agg * (1.0 - ALPHA) + ALPHA * x0_ref[...]
    return out * (1.0 - bl) + bl * jnp.dot(
        out, w_ref[...], preferred_element_type=jnp.float32)


def _layer_body(p_ref, x0_ref, w_ref, g_ref, bta_ref, u_ref, z_ref, *, bl):
    u = _gcn_update(p_ref, x0_ref, w_ref, bl)
    u_ref[...] = u
    mean = jnp.sum(u, axis=0, keepdims=True) * (1.0 / N)
    d = u - mean
    mask = _row_mask()
    d = jnp.where(mask, d, 0.0)
    var = jnp.sum(d * d, axis=0, keepdims=True) * (1.0 / N)
    zn = d * lax.rsqrt(var + 1e-5) * g_ref[...] + bta_ref[...]
    zn = jnp.maximum(zn, 0.0)
    z_ref[...] = jnp.where(mask, zn, 0.0)


def _make_layer(bl):
    return pl.pallas_call(
        functools.partial(_layer_body, bl=bl),
        out_shape=(jax.ShapeDtypeStruct((N_PAD, D), jnp.float32),
                   jax.ShapeDtypeStruct((N_PAD, D), jnp.float32)),
    )


def _jk_body(p_ref, x0_ref, w_ref, z0_ref, z1_ref, z2_ref, wjk_ref, bjk_ref,
             z_ref, *, bl):
    u3 = _gcn_update(p_ref, x0_ref, w_ref, bl)
    acc = jnp.dot(z0_ref[...], wjk_ref[0], preferred_element_type=jnp.float32)
    acc += jnp.dot(z1_ref[...], wjk_ref[1], preferred_element_type=jnp.float32)
    acc += jnp.dot(z2_ref[...], wjk_ref[2], preferred_element_type=jnp.float32)
    acc += jnp.dot(u3, wjk_ref[3], preferred_element_type=jnp.float32)
    acc += bjk_ref[...]
    z_ref[...] = jnp.where(_row_mask(), acc, 0.0)


def _make_jk(bl):
    return pl.pallas_call(
        functools.partial(_jk_body, bl=bl),
        out_shape=jax.ShapeDtypeStruct((N_PAD, D), jnp.float32),
    )


def _final_body(p_ref, x0_ref, w_ref, o_ref, *, bl):
    u = _gcn_update(p_ref, x0_ref, w_ref, bl)
    o_ref[...] = u[:N]


def _make_final(bl):
    return pl.pallas_call(
        functools.partial(_final_body, bl=bl),
        out_shape=jax.ShapeDtypeStruct((N, D), jnp.float32),
    )


def kernel(x, edge_index, W0, b0, Wc, W_jk, b_jk, gamma, beta):
    src = edge_index[0]
    dst = edge_index[1]
    pad = N + (jnp.arange(E_PAD - E, dtype=jnp.int32) % (N_PAD - N))
    src_p = jnp.concatenate([src, pad])
    dst_p = jnp.concatenate([dst, pad])
    pk = ((dst_p << 16) | src_p).reshape(NW, NCHUNK, CHUNK)

    x_p = jnp.zeros((N_PAD, D), jnp.float32).at[:N].set(x)
    b0r = b0.reshape(1, D)
    bjkr = b_jk.reshape(1, D)
    wjk = W_jk.reshape(4, D, D)

    z = _lin0(x_p, W0, b0r)
    x0 = z
    zs = []
    for i in range(L):
        bl = float(math.log(THETA / (i + 1) + 1.0))
        parts = _sc_agg(z, pk)
        if i < L - 2:
            u, z = _make_layer(bl)(parts, x0, Wc[i],
                                   gamma[i].reshape(1, D),
                                   beta[i].reshape(1, D))
            zs.append(u)
        elif i == L - 2:
            z = _make_jk(bl)(parts, x0, Wc[i], zs[0], zs[1], zs[2],
                             wjk, bjkr)
        else:
            z = _make_final(bl)(parts, x0, Wc[i])
    return z


# speedup vs baseline: 1.5387x; 1.1648x over previous
import functools
import math

import jax
import jax.numpy as jnp
from jax import lax
from jax.experimental import pallas as pl
from jax.experimental.pallas import tpu as pltpu
from jax.experimental.pallas import tpu_sc as plsc

N = 10000
E = 320000
D = 128
L = 5
ALPHA = 0.1
THETA = 0.5

NC = 2
NS = 16
NW = NC * NS
CHUNK = 32
NCHUNK = 320
NBUF = 4
EPT = NCHUNK * CHUNK
E_PAD = EPT * NW
ROWS_PER_TILE = 640
N_PAD = NS * ROWS_PER_TILE


def _sc_agg_body(z_hbm, pk_hbm, out_hbm,
                 pk_v, src_c, dst_c, rows_0, rows_1, rows_2, rows_3,
                 agg_sh, sem_0, sem_1, sem_2, sem_3):
    rows = (rows_0, rows_1, rows_2, rows_3)
    sems = (sem_0, sem_1, sem_2, sem_3)
    c = lax.axis_index("c")
    s = lax.axis_index("s")
    wid = c * NS + s

    pltpu.sync_copy(pk_hbm.at[wid], pk_v)

    def unpack(j, b):
        def upk(k, _):
            v = pk_v[j, pl.ds(k * 16, 16)]
            src_c[b, pl.ds(k * 16, 16)] = v & 0xFFFF
            dst_c[b, pl.ds(k * 16, 16)] = lax.shift_right_logical(v, 16)
            return 0
        lax.fori_loop(0, CHUNK // 16, upk, 0)

    zero16 = jnp.zeros((16,), jnp.float32)

    def zrow(i, _):
        def zcol(k, _):
            rows_0[i, pl.ds(k * 16, 16)] = zero16
            return 0
        return lax.fori_loop(0, D // 16, zcol, 0)

    lax.fori_loop(0, CHUNK, zrow, 0)

    row_base = s * ROWS_PER_TILE
    for r in range(ROWS_PER_TILE // CHUNK):
        pltpu.sync_copy(rows_0, agg_sh.at[pl.ds(row_base + r * CHUNK, CHUNK)])
    plsc.subcore_barrier()

    nit = NCHUNK // NBUF
    for b in range(NBUF):
        unpack(b, b)
        pltpu.async_copy(z_hbm.at[src_c.at[b]], rows[b], sems[b])

    def body(i, _):
        j0 = i * NBUF
        for b in range(NBUF):
            pltpu.make_async_copy(z_hbm.at[src_c.at[b]], rows[b],
                                  sems[b]).wait()
            pltpu.sync_copy(rows[b], agg_sh.at[dst_c.at[b]], add=True)

            @pl.when(i + 1 < nit)
            def _(b=b, j0=j0):
                unpack(j0 + NBUF + b, b)
                pltpu.async_copy(z_hbm.at[src_c.at[b]], rows[b], sems[b])

        return 0

    lax.fori_loop(0, nit, body, 0)
    plsc.subcore_barrier()

    pltpu.sync_copy(agg_sh.at[pl.ds(row_base, ROWS_PER_TILE)],
                    out_hbm.at[c, pl.ds(row_base, ROWS_PER_TILE)])


@functools.cache
def _get_sc_agg():
    return functools.partial(
        pl.kernel,
        out_type=jax.ShapeDtypeStruct((NC, N_PAD, D), jnp.float32),
        mesh=plsc.VectorSubcoreMesh(core_axis_name="c", subcore_axis_name="s",
                                    num_cores=NC, num_subcores=NS),
        scratch_types=[
            pltpu.VMEM((NCHUNK, CHUNK), jnp.int32),
            pltpu.VMEM((NBUF, CHUNK), jnp.int32),
            pltpu.VMEM((NBUF, CHUNK), jnp.int32),
            pltpu.VMEM((CHUNK, D), jnp.float32),
            pltpu.VMEM((CHUNK, D), jnp.float32),
            pltpu.VMEM((CHUNK, D), jnp.float32),
            pltpu.VMEM((CHUNK, D), jnp.float32),
            pltpu.VMEM_SHARED((N_PAD, D), jnp.float32),
            pltpu.SemaphoreType.DMA,
            pltpu.SemaphoreType.DMA,
            pltpu.SemaphoreType.DMA,
            pltpu.SemaphoreType.DMA,
        ],
        compiler_params=pltpu.CompilerParams(use_tc_tiling_on_sc=False),
    )(_sc_agg_body)


def _sc_agg(z, pk):
    return _get_sc_agg()(z, pk)


def _row_mask():
    rows = lax.broadcasted_iota(jnp.int32, (N_PAD, 1), 0)
    return rows < N


def _lin0_body(x_ref, w_ref, b_ref, o_ref):
    z = jnp.dot(x_ref[...], w_ref[...], preferred_element_type=jnp.float32)
    z = z + b_ref[...]
    o_ref[...] = jnp.where(_row_mask(), z, 0.0)


_lin0 = pl.pallas_call(
    _lin0_body,
    out_shape=jax.ShapeDtypeStruct((N_PAD, D), jnp.float32),
)


def _gcn_update(p_ref, x0_ref, w_ref, bl):
    agg = p_ref[0] + p_ref[1]
    out = agg * (1.0 - ALPHA) + ALPHA * x0_ref[...]
    return out * (1.0 - bl) + bl * jnp.dot(
        out, w_ref[...], preferred_element_type=jnp.float32)


def _layer_body(p_ref, x0_ref, w_ref, g_ref, bta_ref, u_ref, z_ref, *, bl):
    u = _gcn_update(p_ref, x0_ref, w_ref, bl)
    u_ref[...] = u
    mean = jnp.sum(u, axis=0, keepdims=True) * (1.0 / N)
    d = u - mean
    mask = _row_mask()
    d = jnp.where(mask, d, 0.0)
    var = jnp.sum(d * d, axis=0, keepdims=True) * (1.0 / N)
    zn = d * lax.rsqrt(var + 1e-5) * g_ref[...] + bta_ref[...]
    zn = jnp.maximum(zn, 0.0)
    z_ref[...] = jnp.where(mask, zn, 0.0)


def _make_layer(bl):
    return pl.pallas_call(
        functools.partial(_layer_body, bl=bl),
        out_shape=(jax.ShapeDtypeStruct((N_PAD, D), jnp.float32),
                   jax.ShapeDtypeStruct((N_PAD, D), jnp.float32)),
    )


def _jk_body(p_ref, x0_ref, w_ref, z0_ref, z1_ref, z2_ref, wjk_ref, bjk_ref,
             z_ref, *, bl):
    u3 = _gcn_update(p_ref, x0_ref, w_ref, bl)
    acc = jnp.dot(z0_ref[...], wjk_ref[0], preferred_element_type=jnp.float32)
    acc += jnp.dot(z1_ref[...], wjk_ref[1], preferred_element_type=jnp.float32)
    acc += jnp.dot(z2_ref[...], wjk_ref[2], preferred_element_type=jnp.float32)
    acc += jnp.dot(u3, wjk_ref[3], preferred_element_type=jnp.float32)
    acc += bjk_ref[...]
    z_ref[...] = jnp.where(_row_mask(), acc, 0.0)


def _make_jk(bl):
    return pl.pallas_call(
        functools.partial(_jk_body, bl=bl),
        out_shape=jax.ShapeDtypeStruct((N_PAD, D), jnp.float32),
    )


def _final_body(p_ref, x0_ref, w_ref, o_ref, *, bl):
    u = _gcn_update(p_ref, x0_ref, w_ref, bl)
    o_ref[...] = u[:N]


def _make_final(bl):
    return pl.pallas_call(
        functools.partial(_final_body, bl=bl),
        out_shape=jax.ShapeDtypeStruct((N, D), jnp.float32),
    )


def kernel(x, edge_index, W0, b0, Wc, W_jk, b_jk, gamma, beta):
    src = edge_index[0]
    dst = edge_index[1]
    pad = N + (jnp.arange(E_PAD - E, dtype=jnp.int32) % (N_PAD - N))
    src_p = jnp.concatenate([src, pad])
    dst_p = jnp.concatenate([dst, pad])
    pk = ((dst_p << 16) | src_p).reshape(NW, NCHUNK, CHUNK)

    x_p = jnp.zeros((N_PAD, D), jnp.float32).at[:N].set(x)
    b0r = b0.reshape(1, D)
    bjkr = b_jk.reshape(1, D)
    wjk = W_jk.reshape(4, D, D)

    z = _lin0(x_p, W0, b0r)
    x0 = z
    zs = []
    for i in range(L):
        bl = float(math.log(THETA / (i + 1) + 1.0))
        parts = _sc_agg(z, pk)
        if i < L - 2:
            u, z = _make_layer(bl)(parts, x0, Wc[i],
                                   gamma[i].reshape(1, D),
                                   beta[i].reshape(1, D))
            zs.append(u)
        elif i == L - 2:
            z = _make_jk(bl)(parts, x0, Wc[i], zs[0], zs[1], zs[2],
                             wjk, bjkr)
        else:
            z = _make_final(bl)(parts, x0, Wc[i])
    return z
